# baseline (speedup 1.0000x reference)
import functools

import jax
import jax.numpy as jnp
from jax import lax
from jax.experimental import pallas as pl
from jax.experimental.pallas import tpu as pltpu
from jax.experimental.pallas import tpu_sc as plsc

_Q = 65536
_B = 1024
_D = 128

_NC = 2
_NS = 16
_NW = _NC * _NS
_RPW = _Q // _NW
_SCH = 256
_SNCH = _RPW // _SCH

_MESH = plsc.VectorSubcoreMesh(core_axis_name="c", subcore_axis_name="s")


@functools.partial(
    pl.kernel,
    out_type=jax.ShapeDtypeStruct((_Q, _D), jnp.float32),
    mesh=_MESH,
    scratch_types=(
        pltpu.VMEM((_SCH, _D), jnp.float32),
        pltpu.VMEM((_SCH, _D), jnp.float32),
        pltpu.SemaphoreType.DMA,
    ),
)
def _sc_copy(keys_hbm, old_hbm, buf0, buf1, isem):
    wid = lax.axis_index("s") * _NC + lax.axis_index("c")
    base = wid * _RPW
    bufs = (buf0, buf1)
    fetch = pltpu.async_copy(keys_hbm.at[pl.ds(base, _SCH)], buf0, isem)
    for ci in range(_SNCH):
        b = base + ci * _SCH
        fetch.wait()
        if ci + 1 < _SNCH:
            fetch = pltpu.async_copy(
                keys_hbm.at[pl.ds(b + _SCH, _SCH)], bufs[(ci + 1) % 2], isem)
        pltpu.sync_copy(bufs[ci % 2], old_hbm.at[pl.ds(b, _SCH)])


_SRC = _Q - _B
_CH = 4096
_K = 4
_A = 2
_CHUNKS = []
_r = 0
while _r < _SRC:
    _CHUNKS.append((_r, min(_CH, _SRC - _r)))
    _r += _CH
_NCH = len(_CHUNKS)


def _tc_body(new_ref, keys_ref, upd_ref, hbuf, hsem, *rest):
    bufs, gsems, ssems = rest[:_K], rest[_K:2 * _K], rest[2 * _K:]

    pltpu.make_async_copy(new_ref, hbuf, hsem).start()

    def gather(ci):
        lo, rows = _CHUNKS[ci]
        return pltpu.make_async_copy(
            keys_ref.at[pl.ds(lo, rows)],
            bufs[ci % _K].at[pl.ds(0, rows)], gsems[ci % _K])

    def scatter(ci):
        lo, rows = _CHUNKS[ci]
        return pltpu.make_async_copy(
            bufs[ci % _K].at[pl.ds(0, rows)],
            upd_ref.at[pl.ds(lo + _B, rows)], ssems[ci % _K])

    gathers = {}
    pending = {}
    for ci in range(min(_A, _NCH)):
        gathers[ci] = gather(ci)
        gathers[ci].start()

    hdone = False
    for ci in range(_NCH):
        nf = ci + _A
        if nf < _NCH:
            if nf - _K >= 0:
                pending.pop(nf - _K).wait()
            gathers[nf] = gather(nf)
            gathers[nf].start()
        gathers.pop(ci).wait()
        w = scatter(ci)
        w.start()
        pending[ci] = w
        if not hdone:
            pltpu.make_async_copy(new_ref, hbuf, hsem).wait()
            pltpu.make_async_copy(hbuf, upd_ref.at[pl.ds(0, _B)], hsem).start()
            hdone = True

    pltpu.make_async_copy(hbuf, upd_ref.at[pl.ds(0, _B)], hsem).wait()
    for ci in sorted(pending):
        pending[ci].wait()


def _tc_shift(new_keys, keys):
    return pl.pallas_call(
        _tc_body,
        in_specs=[
            pl.BlockSpec(memory_space=pl.ANY),
            pl.BlockSpec(memory_space=pl.ANY),
        ],
        out_specs=pl.BlockSpec(memory_space=pl.ANY),
        out_shape=jax.ShapeDtypeStruct((_Q, _D), jnp.float32),
        scratch_shapes=(
            [pltpu.VMEM((_B, _D), jnp.float32), pltpu.SemaphoreType.DMA]
            + [pltpu.VMEM((_CH, _D), jnp.float32) for _ in range(_K)]
            + [pltpu.SemaphoreType.DMA for _ in range(2 * _K)]
        ),
    )(new_keys, keys)


def kernel(new_keys, keys):
    old = _sc_copy(keys)
    upd = _tc_shift(new_keys, keys)
    return (old, upd)


# speedup vs baseline: 1.8485x; 1.8485x over previous
import jax
import jax.numpy as jnp
from jax.experimental import pallas as pl
from jax.experimental.pallas import tpu as pltpu

_Q = 65536
_B = 1024
_D = 128
_CH = 8192
_NCH = _Q // _CH
_K = 3
_A = 2


def _body(new_ref, keys_ref, old_ref, upd_ref, hbuf, hsem, *rest):
    bufs, gsems, ssems = rest[:_K], rest[_K:2 * _K], rest[2 * _K:]

    pltpu.make_async_copy(new_ref, hbuf, hsem).start()

    def gather(ci):
        return pltpu.make_async_copy(
            keys_ref.at[pl.ds(ci * _CH, _CH)], bufs[ci % _K], gsems[ci % _K])

    def scatters(ci):
        ws = [pltpu.make_async_copy(
            bufs[ci % _K], old_ref.at[pl.ds(ci * _CH, _CH)], ssems[ci % _K])]
        lo = ci * _CH + _B
        rows = min(_CH, _Q - lo)
        if rows > 0:
            ws.append(pltpu.make_async_copy(
                bufs[ci % _K].at[pl.ds(0, rows)],
                upd_ref.at[pl.ds(lo, rows)], ssems[ci % _K]))
        return ws

    gathers = {}
    pending = {}
    for ci in range(min(_A, _NCH)):
        gathers[ci] = gather(ci)
        gathers[ci].start()

    hdone = False
    for ci in range(_NCH):
        nf = ci + _A
        if nf < _NCH:
            if nf - _K >= 0:
                for w in pending.pop(nf - _K):
                    w.wait()
            gathers[nf] = gather(nf)
            gathers[nf].start()
        gathers.pop(ci).wait()
        ws = scatters(ci)
        for w in ws:
            w.start()
        pending[ci] = ws
        if not hdone:
            pltpu.make_async_copy(new_ref, hbuf, hsem).wait()
            pltpu.make_async_copy(hbuf, upd_ref.at[pl.ds(0, _B)], hsem).start()
            hdone = True

    pltpu.make_async_copy(hbuf, upd_ref.at[pl.ds(0, _B)], hsem).wait()
    for ci in sorted(pending):
        for w in pending[ci]:
            w.wait()


def kernel(new_keys, keys):
    old, upd = pl.pallas_call(
        _body,
        in_specs=[
            pl.BlockSpec(memory_space=pl.ANY),
            pl.BlockSpec(memory_space=pl.ANY),
        ],
        out_specs=[
            pl.BlockSpec(memory_space=pl.ANY),
            pl.BlockSpec(memory_space=pl.ANY),
        ],
        out_shape=[
            jax.ShapeDtypeStruct((_Q, _D), jnp.float32),
            jax.ShapeDtypeStruct((_Q, _D), jnp.float32),
        ],
        scratch_shapes=(
            [pltpu.VMEM((_B, _D), jnp.float32), pltpu.SemaphoreType.DMA]
            + [pltpu.VMEM((_CH, _D), jnp.float32) for _ in range(_K)]
            + [pltpu.SemaphoreType.DMA for _ in range(2 * _K)]
        ),
    )(new_keys, keys)
    return (old, upd)
